# Initial kernel scaffold; baseline (speedup 1.0000x reference)
import jax
import jax.numpy as jnp
from jax.experimental import pallas as pl


def kernel(x, batch, W1, b1, W2, b2):
    raise NotImplementedError("write your pallas kernel here")


# speedup vs baseline: 1.5756x; 1.5756x over previous
import functools

import jax
import jax.numpy as jnp
from jax import lax
from jax.experimental import pallas as pl
from jax.experimental.pallas import tpu as pltpu
from jax.experimental.pallas import tpu_sc as plsc

SEG = 1024
LN = 16
NC = 2
NS = 16
NW = NC * NS
SPW = SEG // NW
CH = 64


def _scores_body(n, x_ref, batch_ref, w1_ref, b1_ref, w2_ref, b2_ref,
                 e_ref, bnd_ref):
    i = pl.program_id(0)
    h = jnp.tanh(
        jnp.dot(x_ref[...], w1_ref[...],
                preferred_element_type=jnp.float32,
                precision=jax.lax.Precision.HIGHEST)
        + b1_ref[...])
    s = jnp.sum(h * w2_ref[...], axis=1) + b2_ref[0]
    e_ref[...] = jnp.exp(s).reshape(e_ref.shape)

    bn = batch_ref.shape[0]
    glob = i * bn + lax.broadcasted_iota(jnp.int32, (bn, 1), 0)
    thresh = SPW * lax.broadcasted_iota(jnp.int32, (1, 128), 1)
    m = (batch_ref[...].reshape(bn, 1) < thresh) & (glob < n)
    partial = jnp.sum(m.astype(jnp.int32), axis=0).reshape(1, 128)

    @pl.when(i == 0)
    def _():
        bnd_ref[...] = jnp.zeros_like(bnd_ref)
    bnd_ref[...] += partial


def _pool_body(n_rows, d, x_hbm, e_hbm, ids_hbm, bnd_hbm, out_hbm,
               xbuf, ebuf, rawid, bbuf, accrow, accsp, table, tsp, obuf):
    c = lax.axis_index("c")
    s = lax.axis_index("s")
    wid = s * NC + c
    seg0 = wid * SPW

    pltpu.sync_copy(bnd_hbm, bbuf)
    bv = bbuf[pl.ds(wid, LN)]
    b_lo = bv[0]
    b_hi = bv[1]

    zero16 = jnp.zeros((LN,), jnp.float32)

    def _ztab(ls, carry):
        for k in range(d // LN):
            table[ls, pl.ds(k * LN, LN)] = zero16
        tsp[ls, pl.ds(0, LN)] = zero16
        return carry
    lax.fori_loop(0, SPW, _ztab, 0)
    for k in range(d // LN):
        accrow[pl.ds(k * LN, LN)] = zero16
    accsp[pl.ds(0, LN)] = zero16

    base = (b_lo // 8) * 8
    nch = (b_hi - base + CH - 1) // CH
    n_clamp = ((n_rows - CH) // 8) * 8

    def _chunk(i, cur):
        lo = base + i * CH
        start = jnp.minimum(lo, n_clamp)
        pltpu.sync_copy(x_hbm.at[pl.ds(start, CH)], xbuf)
        pltpu.sync_copy(e_hbm.at[pl.ds(start, CH)], ebuf.at[pl.ds(0, CH)])
        pltpu.sync_copy(ids_hbm.at[pl.ds(start, CH)], rawid.at[pl.ds(0, CH)])

        def _row(j, cur2):
            g = start + j
            valid = (g >= b_lo) & (g >= lo) & (g < b_hi) & (g < lo + CH)
            id_j = rawid[pl.ds(j, LN)][0]
            lseg = jnp.clip(id_j - seg0, 0, SPW - 1)

            @pl.when(lseg != cur2)
            def _flush():
                for k in range(d // LN):
                    sl = pl.ds(k * LN, LN)
                    table[cur2, sl] += accrow[sl]
                    accrow[sl] = zero16
                tsp[cur2, pl.ds(0, LN)] += accsp[pl.ds(0, LN)]
                accsp[pl.ds(0, LN)] = zero16

            evs = jnp.where(valid, ebuf[pl.ds(j, LN)][0], 0.0)
            evv = jnp.full((LN,), evs)
            for k in range(d // LN):
                sl = pl.ds(k * LN, LN)
                accrow[sl] += xbuf[j, sl] * evv
            lane = lax.iota(jnp.int32, LN)
            sp = jnp.where(lane == 0, evv,
                           jnp.where(lane == 1,
                                     jnp.where(valid, jnp.float32(1.0),
                                               jnp.float32(0.0)),
                                     jnp.float32(0.0)))
            accsp[pl.ds(0, LN)] += sp
            return lseg
        return lax.fori_loop(0, CH, _row, cur)
    cur_end = lax.fori_loop(0, nch, _chunk, 0)

    for k in range(d // LN):
        sl = pl.ds(k * LN, LN)
        table[cur_end, sl] += accrow[sl]
    tsp[cur_end, pl.ds(0, LN)] += accsp[pl.ds(0, LN)]

    def _norm(ls, carry):
        spv = tsp[ls, pl.ds(0, LN)]
        se = spv[0]
        cnt = spv[1]
        denv = jnp.full((LN,), (se + 1e-16) * jnp.maximum(cnt, 1.0))
        rinv = jnp.float32(1.0) / denv
        for k in range(d // LN):
            sl = pl.ds(k * LN, LN)
            obuf[ls, sl] = table[ls, sl] * rinv
        return carry
    lax.fori_loop(0, SPW, _norm, 0)
    pltpu.sync_copy(obuf, out_hbm.at[pl.ds(seg0, SPW)])


def kernel(x, batch, W1, b1, W2, b2):
    n, d = x.shape
    assert n % 8 == 0 and d % LN == 0 and n >= CH

    bn = 1024
    nblk = (n + bn - 1) // bn
    e2, bnd = pl.pallas_call(
        functools.partial(_scores_body, n),
        grid=(nblk,),
        in_specs=[
            pl.BlockSpec((bn, d), lambda i: (i, 0)),
            pl.BlockSpec((bn,), lambda i: (i,)),
            pl.BlockSpec((d, d), lambda i: (0, 0)),
            pl.BlockSpec((1, d), lambda i: (0, 0)),
            pl.BlockSpec((1, d), lambda i: (0, 0)),
            pl.BlockSpec(memory_space=pltpu.SMEM),
        ],
        out_specs=[
            pl.BlockSpec((1, 1, bn), lambda i: (i, 0, 0)),
            pl.BlockSpec((1, 128), lambda i: (0, 0)),
        ],
        out_shape=[
            jax.ShapeDtypeStruct((nblk, 1, bn), jnp.float32),
            jax.ShapeDtypeStruct((1, 128), jnp.int32),
        ],
        compiler_params=pltpu.CompilerParams(
            dimension_semantics=("arbitrary",)),
    )(x, batch, W1, b1.reshape(1, d), W2.reshape(1, d), b2)
    e = e2.reshape(-1)[:n]

    mesh = plsc.VectorSubcoreMesh(core_axis_name="c", subcore_axis_name="s")
    out = pl.kernel(
        functools.partial(_pool_body, n, d),
        mesh=mesh,
        out_type=jax.ShapeDtypeStruct((SEG, d), jnp.float32),
        scratch_types=[
            pltpu.VMEM((CH, d), jnp.float32),
            pltpu.VMEM((CH + LN,), jnp.float32),
            pltpu.VMEM((CH + LN,), jnp.int32),
            pltpu.VMEM((128,), jnp.int32),
            pltpu.VMEM((d,), jnp.float32),
            pltpu.VMEM((LN,), jnp.float32),
            pltpu.VMEM((SPW, d), jnp.float32),
            pltpu.VMEM((SPW, LN), jnp.float32),
            pltpu.VMEM((SPW, d), jnp.float32),
        ],
    )(x, e, batch, bnd.reshape(128))
    return out


# speedup vs baseline: 3.2484x; 2.0618x over previous
import functools

import jax
import jax.numpy as jnp
from jax import lax
from jax.experimental import pallas as pl
from jax.experimental.pallas import tpu as pltpu
from jax.experimental.pallas import tpu_sc as plsc

SEG = 1024
LN = 16
NC = 2
NS = 16
NW = NC * NS
SPW = SEG // NW
CH = 64


def _scores_body(n, x_ref, batch_ref, w1_ref, b1_ref, w2_ref, b2_ref,
                 e_ref, bnd_ref):
    i = pl.program_id(0)
    h = jnp.tanh(
        jnp.dot(x_ref[...], w1_ref[...],
                preferred_element_type=jnp.float32,
                precision=jax.lax.Precision.HIGHEST)
        + b1_ref[...])
    s = jnp.sum(h * w2_ref[...], axis=1) + b2_ref[0]
    e_ref[...] = jnp.exp(s).reshape(e_ref.shape)

    bn = batch_ref.shape[0]
    glob = i * bn + lax.broadcasted_iota(jnp.int32, (bn, 1), 0)
    thresh = SPW * lax.broadcasted_iota(jnp.int32, (1, 128), 1)
    m = (batch_ref[...].reshape(bn, 1) < thresh) & (glob < n)
    partial = jnp.sum(m.astype(jnp.int32), axis=0).reshape(1, 128)

    @pl.when(i == 0)
    def _():
        bnd_ref[...] = jnp.zeros_like(bnd_ref)
    bnd_ref[...] += partial


def _pool_body(n_rows, d, x_hbm, e_hbm, ids_hbm, bnd_hbm, out_hbm,
               xbuf, ebuf, rawid, bbuf, table, tsp, sem):
    nk = d // LN
    c = lax.axis_index("c")
    s = lax.axis_index("s")
    wid = s * NC + c
    seg0 = wid * SPW
    zero16 = jnp.zeros((LN,), jnp.float32)
    lane = lax.iota(jnp.int32, LN)

    pltpu.sync_copy(bnd_hbm, bbuf)
    bv = bbuf[pl.ds(wid, LN)]
    b_lo = bv[0]
    b_hi = bv[1]

    def _ztab(ls, carry):
        for k in range(nk):
            table[ls, pl.ds(k * LN, LN)] = zero16
        tsp[ls, pl.ds(0, LN)] = zero16
        return carry
    lax.fori_loop(0, SPW, _ztab, 0)

    base = (b_lo // 8) * 8
    nch = (b_hi - base + CH - 1) // CH
    n_clamp = ((n_rows - CH) // 8) * 8

    def _dma_args(ci, slot):
        st = jnp.minimum(base + ci * CH, n_clamp)
        return (
            (x_hbm.at[pl.ds(st, CH)], xbuf.at[slot]),
            (e_hbm.at[pl.ds(st, CH)], ebuf.at[slot].at[pl.ds(0, CH)]),
            (ids_hbm.at[pl.ds(st, CH)], rawid.at[slot].at[pl.ds(0, CH)]),
        )

    def _issue(ci, slot):
        for src, dst in _dma_args(ci, slot):
            pltpu.async_copy(src, dst, sem.at[slot])

    def _drain(ci, slot):
        for src, dst in _dma_args(ci, slot):
            pltpu.make_async_copy(src, dst, sem.at[slot]).wait()

    @pl.when(nch > 0)
    def _():
        _issue(0, 0)

    def _chunk(i, carry):
        slot = lax.rem(i, 2)
        lo = base + i * CH
        start = jnp.minimum(lo, n_clamp)
        _drain(i, slot)

        @pl.when(i + 1 < nch)
        def _():
            _issue(i + 1, 1 - slot)

        def _row(j, carry2):
            cur, accsp, accs = carry2
            g = start + j
            valid = (g >= b_lo) & (g >= lo) & (g < b_hi) & (g < lo + CH)
            id_j = rawid[slot, pl.ds(j, LN)][0]
            lseg = jnp.clip(id_j - seg0, 0, SPW - 1)
            changed = lseg != cur

            @pl.when(changed)
            def _flush():
                for k in range(nk):
                    sl = pl.ds(k * LN, LN)
                    table[cur, sl] += accs[k]
                tsp[cur, pl.ds(0, LN)] += accsp

            keep = jnp.full(
                (LN,), jnp.where(changed, jnp.float32(0.0), jnp.float32(1.0)))
            evs = jnp.where(valid, ebuf[slot, pl.ds(j, LN)][0], 0.0)
            evv = jnp.full((LN,), evs)
            new_accs = tuple(
                accs[k] * keep + xbuf[slot, j, pl.ds(k * LN, LN)] * evv
                for k in range(nk))
            sp = jnp.where(lane == 0, evv,
                           jnp.where(lane == 1,
                                     jnp.where(valid, jnp.float32(1.0),
                                               jnp.float32(0.0)),
                                     jnp.float32(0.0)))
            return (lseg, accsp * keep + sp, new_accs)
        return lax.fori_loop(0, CH, _row, carry)

    init = (jnp.int32(0), zero16, tuple(zero16 for _ in range(nk)))
    cur_end, accsp_end, accs_end = lax.fori_loop(0, nch, _chunk, init)

    for k in range(nk):
        table[cur_end, pl.ds(k * LN, LN)] += accs_end[k]
    tsp[cur_end, pl.ds(0, LN)] += accsp_end

    def _norm(ls, carry):
        spv = tsp[ls, pl.ds(0, LN)]
        denv = jnp.full((LN,), spv[0] + 1e-16) * \
            jnp.maximum(jnp.full((LN,), spv[1]), 1.0)
        rinv = jnp.float32(1.0) / denv
        for k in range(nk):
            sl = pl.ds(k * LN, LN)
            table[ls, sl] = table[ls, sl] * rinv
        return carry
    lax.fori_loop(0, SPW, _norm, 0)
    pltpu.sync_copy(table, out_hbm.at[pl.ds(seg0, SPW)])


def kernel(x, batch, W1, b1, W2, b2):
    n, d = x.shape
    assert n % 8 == 0 and d % LN == 0 and n >= CH

    bn = 1024
    nblk = (n + bn - 1) // bn
    e2, bnd = pl.pallas_call(
        functools.partial(_scores_body, n),
        grid=(nblk,),
        in_specs=[
            pl.BlockSpec((bn, d), lambda i: (i, 0)),
            pl.BlockSpec((bn,), lambda i: (i,)),
            pl.BlockSpec((d, d), lambda i: (0, 0)),
            pl.BlockSpec((1, d), lambda i: (0, 0)),
            pl.BlockSpec((1, d), lambda i: (0, 0)),
            pl.BlockSpec(memory_space=pltpu.SMEM),
        ],
        out_specs=[
            pl.BlockSpec((1, 1, bn), lambda i: (i, 0, 0)),
            pl.BlockSpec((1, 128), lambda i: (0, 0)),
        ],
        out_shape=[
            jax.ShapeDtypeStruct((nblk, 1, bn), jnp.float32),
            jax.ShapeDtypeStruct((1, 128), jnp.int32),
        ],
        compiler_params=pltpu.CompilerParams(
            dimension_semantics=("arbitrary",)),
    )(x, batch, W1, b1.reshape(1, d), W2.reshape(1, d), b2)
    e = e2.reshape(-1)[:n]

    mesh = plsc.VectorSubcoreMesh(core_axis_name="c", subcore_axis_name="s")
    out = pl.kernel(
        functools.partial(_pool_body, n, d),
        mesh=mesh,
        out_type=jax.ShapeDtypeStruct((SEG, d), jnp.float32),
        scratch_types=[
            pltpu.VMEM((2, CH, d), jnp.float32),
            pltpu.VMEM((2, CH + LN), jnp.float32),
            pltpu.VMEM((2, CH + LN), jnp.int32),
            pltpu.VMEM((128,), jnp.int32),
            pltpu.VMEM((SPW, d), jnp.float32),
            pltpu.VMEM((SPW, LN), jnp.float32),
            pltpu.SemaphoreType.DMA((2,)),
        ],
    )(x, e, batch, bnd.reshape(128))
    return out


# speedup vs baseline: 3.3751x; 1.0390x over previous
import functools

import jax
import jax.numpy as jnp
from jax import lax
from jax.experimental import pallas as pl
from jax.experimental.pallas import tpu as pltpu
from jax.experimental.pallas import tpu_sc as plsc

SEG = 1024
LN = 16
NC = 2
NS = 16
NW = NC * NS
SPW = SEG // NW
CH = 64


def _scores_body(n, x_ref, batch_ref, w1_ref, b1_ref, w2_ref, b2_ref,
                 e_ref, bnd_ref):
    i = pl.program_id(0)
    h = jnp.tanh(
        jnp.dot(x_ref[...], w1_ref[...],
                preferred_element_type=jnp.float32,
                precision=jax.lax.Precision.DEFAULT)
        + b1_ref[...])
    s = jnp.sum(h * w2_ref[...], axis=1) + b2_ref[0]
    e_ref[...] = s.reshape(e_ref.shape)

    bn = batch_ref.shape[0]
    glob = i * bn + lax.broadcasted_iota(jnp.int32, (bn, 1), 0)
    thresh = SPW * lax.broadcasted_iota(jnp.int32, (1, 128), 1)
    m = (batch_ref[...].reshape(bn, 1) < thresh) & (glob < n)
    partial = jnp.sum(m.astype(jnp.int32), axis=0).reshape(1, 128)

    @pl.when(i == 0)
    def _():
        bnd_ref[...] = jnp.zeros_like(bnd_ref)
    bnd_ref[...] += partial


def _pool_body(n_rows, d, x_hbm, e_hbm, ids_hbm, bnd_hbm, out_hbm,
               xbuf, ebuf, rawid, bbuf, table, tsp, sem):
    nk = d // LN
    c = lax.axis_index("c")
    s = lax.axis_index("s")
    wid = s * NC + c
    seg0 = wid * SPW
    zero16 = jnp.zeros((LN,), jnp.float32)
    lane = lax.iota(jnp.int32, LN)

    pltpu.sync_copy(bnd_hbm, bbuf)
    bv = bbuf[pl.ds(wid, LN)]
    b_lo = bv[0]
    b_hi = bv[1]

    def _ztab(ls, carry):
        for k in range(nk):
            table[ls, pl.ds(k * LN, LN)] = zero16
        tsp[ls, pl.ds(0, LN)] = zero16
        return carry
    lax.fori_loop(0, SPW, _ztab, 0)

    base = (b_lo // 8) * 8
    nch = (b_hi - base + CH - 1) // CH
    n_clamp = ((n_rows - CH) // 8) * 8

    def _dma_args(ci, slot):
        st = jnp.minimum(base + ci * CH, n_clamp)
        return (
            (x_hbm.at[pl.ds(st, CH)], xbuf.at[slot]),
            (e_hbm.at[pl.ds(st, CH)], ebuf.at[slot].at[pl.ds(0, CH)]),
            (ids_hbm.at[pl.ds(st, CH)], rawid.at[slot].at[pl.ds(0, CH)]),
        )

    def _issue(ci, slot):
        for src, dst in _dma_args(ci, slot):
            pltpu.async_copy(src, dst, sem.at[slot])

    def _drain(ci, slot):
        for src, dst in _dma_args(ci, slot):
            pltpu.make_async_copy(src, dst, sem.at[slot]).wait()

    @pl.when(nch > 0)
    def _():
        _issue(0, 0)

    def _chunk(i, carry):
        slot = lax.rem(i, 2)
        lo = base + i * CH
        start = jnp.minimum(lo, n_clamp)
        _drain(i, slot)

        @pl.when(i + 1 < nch)
        def _():
            _issue(i + 1, 1 - slot)

        def _grp(j16, carry2):
            cur, accsp, accs = carry2
            gbase = j16 * LN
            ev16 = jnp.exp(ebuf[slot, pl.ds(gbase, LN)])
            idv16 = rawid[slot, pl.ds(gbase, LN)]
            for rr in range(LN):
                j = gbase + rr
                g = start + j
                valid = (g >= b_lo) & (g >= lo) & (g < b_hi) & (g < lo + CH)
                id_j = idv16[rr]
                lseg = jnp.clip(id_j - seg0, 0, SPW - 1)
                changed = lseg != cur
                cur_old, accsp_old, accs_old = cur, accsp, accs

                @pl.when(changed)
                def _flush(cur_old=cur_old, accsp_old=accsp_old,
                           accs_old=accs_old):
                    for k in range(nk):
                        sl = pl.ds(k * LN, LN)
                        table[cur_old, sl] += accs_old[k]
                    tsp[cur_old, pl.ds(0, LN)] += accsp_old

                keep = jnp.full(
                    (LN,),
                    jnp.where(changed, jnp.float32(0.0), jnp.float32(1.0)))
                evs = jnp.where(valid, ev16[rr], 0.0)
                evv = jnp.full((LN,), evs)
                accs = tuple(
                    accs[k] * keep + xbuf[slot, j, pl.ds(k * LN, LN)] * evv
                    for k in range(nk))
                sp = jnp.where(lane == 0, evv,
                               jnp.where(lane == 1,
                                         jnp.where(valid, jnp.float32(1.0),
                                                   jnp.float32(0.0)),
                                         jnp.float32(0.0)))
                accsp = accsp * keep + sp
                cur = lseg
            return (cur, accsp, accs)
        return lax.fori_loop(0, CH // LN, _grp, carry)

    init = (jnp.int32(0), zero16, tuple(zero16 for _ in range(nk)))
    cur_end, accsp_end, accs_end = lax.fori_loop(0, nch, _chunk, init)

    for k in range(nk):
        table[cur_end, pl.ds(k * LN, LN)] += accs_end[k]
    tsp[cur_end, pl.ds(0, LN)] += accsp_end

    def _norm(ls, carry):
        spv = tsp[ls, pl.ds(0, LN)]
        denv = jnp.full((LN,), spv[0] + 1e-16) * \
            jnp.maximum(jnp.full((LN,), spv[1]), 1.0)
        rinv = jnp.float32(1.0) / denv
        for k in range(nk):
            sl = pl.ds(k * LN, LN)
            table[ls, sl] = table[ls, sl] * rinv
        return carry
    lax.fori_loop(0, SPW, _norm, 0)
    pltpu.sync_copy(table, out_hbm.at[pl.ds(seg0, SPW)])


def kernel(x, batch, W1, b1, W2, b2):
    n, d = x.shape
    assert n % 8 == 0 and d % LN == 0 and n >= CH

    bn = 1024
    nblk = (n + bn - 1) // bn
    e2, bnd = pl.pallas_call(
        functools.partial(_scores_body, n),
        grid=(nblk,),
        in_specs=[
            pl.BlockSpec((bn, d), lambda i: (i, 0)),
            pl.BlockSpec((bn,), lambda i: (i,)),
            pl.BlockSpec((d, d), lambda i: (0, 0)),
            pl.BlockSpec((1, d), lambda i: (0, 0)),
            pl.BlockSpec((1, d), lambda i: (0, 0)),
            pl.BlockSpec(memory_space=pltpu.SMEM),
        ],
        out_specs=[
            pl.BlockSpec((1, 1, bn), lambda i: (i, 0, 0)),
            pl.BlockSpec((1, 128), lambda i: (0, 0)),
        ],
        out_shape=[
            jax.ShapeDtypeStruct((nblk, 1, bn), jnp.float32),
            jax.ShapeDtypeStruct((1, 128), jnp.int32),
        ],
        compiler_params=pltpu.CompilerParams(
            dimension_semantics=("arbitrary",)),
    )(x, batch, W1, b1.reshape(1, d), W2.reshape(1, d), b2)
    e = e2.reshape(-1)[:n]

    mesh = plsc.VectorSubcoreMesh(core_axis_name="c", subcore_axis_name="s")
    out = pl.kernel(
        functools.partial(_pool_body, n, d),
        mesh=mesh,
        out_type=jax.ShapeDtypeStruct((SEG, d), jnp.float32),
        scratch_types=[
            pltpu.VMEM((2, CH, d), jnp.float32),
            pltpu.VMEM((2, CH + LN), jnp.float32),
            pltpu.VMEM((2, CH + LN), jnp.int32),
            pltpu.VMEM((128,), jnp.int32),
            pltpu.VMEM((SPW, d), jnp.float32),
            pltpu.VMEM((SPW, LN), jnp.float32),
            pltpu.SemaphoreType.DMA((2,)),
        ],
    )(x, e, batch, bnd.reshape(128))
    return out


# speedup vs baseline: 5.5967x; 1.6582x over previous
import functools

import jax
import jax.numpy as jnp
from jax import lax
from jax.experimental import pallas as pl
from jax.experimental.pallas import tpu as pltpu
from jax.experimental.pallas import tpu_sc as plsc

SEG = 1024
LN = 16
NC = 2
NS = 16
NW = NC * NS
SPW = SEG // NW
CH = 64


def _scores_body(n, x_ref, batch_ref, w1_ref, b1_ref, w2_ref, b2_ref,
                 e_ref, bnd_ref):
    i = pl.program_id(0)
    h = jnp.tanh(
        jnp.dot(x_ref[...], w1_ref[...],
                preferred_element_type=jnp.float32,
                precision=jax.lax.Precision.DEFAULT)
        + b1_ref[...])
    s = jnp.sum(h * w2_ref[...], axis=1) + b2_ref[0]
    e_ref[...] = s.reshape(e_ref.shape)

    bn = batch_ref.shape[0]
    glob = i * bn + lax.broadcasted_iota(jnp.int32, (bn, 1), 0)
    thresh = SPW * lax.broadcasted_iota(jnp.int32, (1, 128), 1)
    m = (batch_ref[...].reshape(bn, 1) < thresh) & (glob < n)
    partial = jnp.sum(m.astype(jnp.int32), axis=0).reshape(1, 128)

    @pl.when(i == 0)
    def _():
        bnd_ref[...] = jnp.zeros_like(bnd_ref)
    bnd_ref[...] += partial


def _pool_body(n_rows, d, x_hbm, e_hbm, ids_hbm, bnd_hbm, out_hbm,
               xbuf, ebuf, rawid, bbuf, table, tsp, sem):
    nk = d // LN
    c = lax.axis_index("c")
    s = lax.axis_index("s")
    wid = s * NC + c
    seg0 = wid * SPW
    zero16 = jnp.zeros((LN,), jnp.float32)
    lane = lax.iota(jnp.int32, LN)

    pltpu.sync_copy(bnd_hbm, bbuf)
    bv = bbuf[pl.ds(wid, LN)]
    b_lo = bv[0]
    b_hi = bv[1]

    def _ztab(ls, carry):
        for k in range(nk):
            table[ls, pl.ds(k * LN, LN)] = zero16
        tsp[ls, pl.ds(0, LN)] = zero16
        return carry
    lax.fori_loop(0, SPW, _ztab, 0)

    base = (b_lo // 8) * 8
    nch = (b_hi - base + CH - 1) // CH
    n_clamp = ((n_rows - CH) // 8) * 8

    def _dma_args(ci, slot):
        st = jnp.minimum(base + ci * CH, n_clamp)
        return (
            (x_hbm.at[pl.ds(st, CH)], xbuf.at[slot]),
            (e_hbm.at[pl.ds(st, CH)], ebuf.at[slot].at[pl.ds(0, CH)]),
            (ids_hbm.at[pl.ds(st, CH)], rawid.at[slot].at[pl.ds(0, CH)]),
        )

    def _issue(ci, slot):
        for src, dst in _dma_args(ci, slot):
            pltpu.async_copy(src, dst, sem.at[slot])

    def _drain(ci, slot):
        for src, dst in _dma_args(ci, slot):
            pltpu.make_async_copy(src, dst, sem.at[slot]).wait()

    @pl.when(nch > 0)
    def _():
        _issue(0, 0)

    def _chunk(i, carry):
        slot = lax.rem(i, 2)
        lo = base + i * CH
        start = jnp.minimum(lo, n_clamp)
        _drain(i, slot)

        @pl.when(i + 1 < nch)
        def _():
            _issue(i + 1, 1 - slot)

        def _row(j, carry2):
            cur, accsp, accs = carry2
            g = start + j
            valid = (g >= b_lo) & (g >= lo) & (g < b_hi) & (g < lo + CH)
            id_j = rawid[slot, pl.ds(j, LN)][0]
            lseg = jnp.clip(id_j - seg0, 0, SPW - 1)
            changed = lseg != cur

            @pl.when(changed)
            def _flush():
                for k in range(nk):
                    sl = pl.ds(k * LN, LN)
                    table[cur, sl] += accs[k]
                tsp[cur, pl.ds(0, LN)] += accsp

            keep = jnp.full(
                (LN,), jnp.where(changed, jnp.float32(0.0), jnp.float32(1.0)))
            vf = jnp.full(
                (LN,), jnp.where(valid, jnp.float32(1.0), jnp.float32(0.0)))
            evv = jnp.exp(jnp.full((LN,), ebuf[slot, pl.ds(j, LN)][0])) * vf
            new_accs = tuple(
                accs[k] * keep + xbuf[slot, j, pl.ds(k * LN, LN)] * evv
                for k in range(nk))
            sp = jnp.where(lane == 0, evv, jnp.where(lane == 1, vf, 0.0))
            return (lseg, accsp * keep + sp, new_accs)
        return lax.fori_loop(0, CH, _row, carry)

    init = (jnp.int32(0), zero16, tuple(zero16 for _ in range(nk)))
    cur_end, accsp_end, accs_end = lax.fori_loop(0, nch, _chunk, init)

    for k in range(nk):
        table[cur_end, pl.ds(k * LN, LN)] += accs_end[k]
    tsp[cur_end, pl.ds(0, LN)] += accsp_end

    def _norm(ls, carry):
        spv = tsp[ls, pl.ds(0, LN)]
        denv = jnp.full((LN,), spv[0] + 1e-16) * \
            jnp.maximum(jnp.full((LN,), spv[1]), 1.0)
        rinv = jnp.float32(1.0) / denv
        for k in range(nk):
            sl = pl.ds(k * LN, LN)
            table[ls, sl] = table[ls, sl] * rinv
        return carry
    lax.fori_loop(0, SPW, _norm, 0)
    pltpu.sync_copy(table, out_hbm.at[pl.ds(seg0, SPW)])


def kernel(x, batch, W1, b1, W2, b2):
    n, d = x.shape
    assert n % 8 == 0 and d % LN == 0 and n >= CH

    bn = 1024
    nblk = (n + bn - 1) // bn
    e2, bnd = pl.pallas_call(
        functools.partial(_scores_body, n),
        grid=(nblk,),
        in_specs=[
            pl.BlockSpec((bn, d), lambda i: (i, 0)),
            pl.BlockSpec((bn,), lambda i: (i,)),
            pl.BlockSpec((d, d), lambda i: (0, 0)),
            pl.BlockSpec((1, d), lambda i: (0, 0)),
            pl.BlockSpec((1, d), lambda i: (0, 0)),
            pl.BlockSpec(memory_space=pltpu.SMEM),
        ],
        out_specs=[
            pl.BlockSpec((1, 1, bn), lambda i: (i, 0, 0)),
            pl.BlockSpec((1, 128), lambda i: (0, 0)),
        ],
        out_shape=[
            jax.ShapeDtypeStruct((nblk, 1, bn), jnp.float32),
            jax.ShapeDtypeStruct((1, 128), jnp.int32),
        ],
        compiler_params=pltpu.CompilerParams(
            dimension_semantics=("arbitrary",)),
    )(x, batch, W1, b1.reshape(1, d), W2.reshape(1, d), b2)
    e = e2.reshape(-1)[:n]

    mesh = plsc.VectorSubcoreMesh(core_axis_name="c", subcore_axis_name="s")
    out = pl.kernel(
        functools.partial(_pool_body, n, d),
        mesh=mesh,
        out_type=jax.ShapeDtypeStruct((SEG, d), jnp.float32),
        scratch_types=[
            pltpu.VMEM((2, CH, d), jnp.float32),
            pltpu.VMEM((2, CH + LN), jnp.float32),
            pltpu.VMEM((2, CH + LN), jnp.int32),
            pltpu.VMEM((128,), jnp.int32),
            pltpu.VMEM((SPW, d), jnp.float32),
            pltpu.VMEM((SPW, LN), jnp.float32),
            pltpu.SemaphoreType.DMA((2,)),
        ],
    )(x, e, batch, bnd.reshape(128))
    return out
